# baseline (speedup 1.0000x reference)
import jax
import jax.numpy as jnp
from jax.experimental import pallas as pl
from jax.experimental.pallas import tpu as pltpu

N = 1000
K = 20000
KB = 2048
NK = (K + KB - 1) // KB
RB = 128
NCH = (N + RB - 1) // RB
SUB = 2
NMS_THR = 0.75


def _body(labels_ref, masks_ref, out_ref, inter_ref, sig_ref):
    k = pl.program_id(0)

    @pl.when(k == 0)
    def _init():
        inter_ref[...] = jnp.zeros_like(inter_ref)
        sig_ref[...] = jnp.zeros_like(sig_ref)

    def _accum(x):
        one = jnp.bfloat16(1.0)
        zero = jnp.bfloat16(0.0)
        parts = [None] * NCH
        for s in range(SUB):
            xs = x[:, s * (KB // SUB):(s + 1) * (KB // SUB)]
            binb = jnp.where(xs.astype(jnp.bfloat16) > zero, one, zero)
            for i in range(NCH):
                r0 = RB * i
                r1 = min(r0 + RB, N)
                d = jax.lax.dot_general(
                    binb[r0:], binb[r0:r1], (((1,), (1,)), ((), ())),
                    preferred_element_type=jnp.float32)
                parts[i] = d if s == 0 else parts[i] + d
        for i in range(NCH):
            r0 = RB * i
            r1 = min(r0 + RB, N)
            inter_ref[r0:, r0:r1] += parts[i]
        th = jnp.maximum(jnp.tanh(x.astype(jnp.bfloat16) * jnp.bfloat16(0.5)),
                         zero)
        sig_ref[...] += jnp.sum(th.astype(jnp.float32), axis=1, keepdims=True)

    @pl.when(k < NK - 1)
    def _steady():
        _accum(masks_ref[...])

    @pl.when(k == NK - 1)
    def _tail():
        col = jax.lax.broadcasted_iota(jnp.int32, (N, KB), 1) + k * KB
        _accum(jnp.where(col < K, masks_ref[...], -1.0))

    @pl.when(k == NK - 1)
    def _finish():
        ri = jax.lax.broadcasted_iota(jnp.int32, (N, N), 0)
        ci = jax.lax.broadcasted_iota(jnp.int32, (N, N), 1)
        ut = inter_ref[...]
        stored = (ri // RB) >= (ci // RB)
        inter = jnp.where(stored, ut, ut.T)

        on_diag = ri == ci
        diag = jnp.where(on_diag, inter, 0.0)
        pn_col = jnp.sum(diag, axis=1, keepdims=True)
        pn_row = jnp.sum(diag, axis=0, keepdims=True)

        labels = labels_ref[...]
        m = jnp.max(labels, axis=1, keepdims=True)
        e = jnp.exp(labels - m)
        p = e / jnp.sum(e, axis=1, keepdims=True)
        li = jax.lax.broadcasted_iota(jnp.int32, (N, 19), 1)
        nms = jnp.max(jnp.where(li < 18, p, -1.0), axis=1, keepdims=True)
        nms = jnp.where(pn_col == 0.0, 0.0, nms)

        conflict = inter * (1.0 + NMS_THR) > NMS_THR * (pn_col + pn_row + 1e-6)

        s_col = nms
        s_row = jnp.sum(jnp.where(on_diag, s_col, 0.0), axis=0, keepdims=True)
        earlier = (s_row > s_col) | ((s_row == s_col) & (ci < ri))
        c_f = jnp.where(conflict & earlier, 1.0, 0.0)

        def cond(carry):
            _, changed, t = carry
            return changed & (t < N + 2)

        def body(carry):
            keep, _, t = carry
            sup = jax.lax.dot_general(
                c_f, keep, (((1,), (0,)), ((), ())),
                preferred_element_type=jnp.float32)
            new = jnp.where(sup > 0.5, 0.0, 1.0)
            changed = jnp.sum(jnp.abs(new - keep)) > 0.0
            return new, changed, t + 1

        keep0 = jnp.ones((N, 1), jnp.float32)
        keep, _, _ = jax.lax.while_loop(
            cond, body, (keep0, jnp.bool_(True), jnp.int32(0)))

        mask_scores = (0.5 * pn_col + 0.5 * sig_ref[...]) / (pn_col + 1e-6)
        out_ref[...] = nms * mask_scores * keep


def kernel(pred_labels, pred_masks):
    out = pl.pallas_call(
        _body,
        grid=(NK,),
        in_specs=[
            pl.BlockSpec((N, 19), lambda k: (0, 0)),
            pl.BlockSpec((N, KB), lambda k: (0, k)),
        ],
        out_specs=pl.BlockSpec((N, 1), lambda k: (0, 0)),
        out_shape=jax.ShapeDtypeStruct((N, 1), jnp.float32),
        scratch_shapes=[
            pltpu.VMEM((N, N), jnp.float32),
            pltpu.VMEM((N, 1), jnp.float32),
        ],
        compiler_params=pltpu.CompilerParams(
            vmem_limit_bytes=100 * 1024 * 1024),
    )(pred_labels, pred_masks)
    return out.reshape(N)


# speedup vs baseline: 1.4844x; 1.4844x over previous
import jax
import jax.numpy as jnp
from jax.experimental import pallas as pl
from jax.experimental.pallas import tpu as pltpu

N = 1000
K = 20000
KB = 2048
NK = (K + KB - 1) // KB
RB = 256
NCH = (N + RB - 1) // RB
SUB = 2
NMS_THR = 0.75


def _body(labels_ref, masks_ref, out_ref, inter_ref, sig_ref):
    k = pl.program_id(0)

    @pl.when(k == 0)
    def _init():
        inter_ref[...] = jnp.zeros_like(inter_ref)
        sig_ref[...] = jnp.zeros_like(sig_ref)

    def _accum(x):
        one = jnp.bfloat16(1.0)
        zero = jnp.bfloat16(0.0)
        parts = [None] * NCH
        for s in range(SUB):
            xs = x[:, s * (KB // SUB):(s + 1) * (KB // SUB)]
            binb = jnp.where(xs.astype(jnp.bfloat16) > zero, one, zero)
            for i in range(NCH):
                r0 = RB * i
                r1 = min(r0 + RB, N)
                d = jax.lax.dot_general(
                    binb[r0:], binb[r0:r1], (((1,), (1,)), ((), ())),
                    preferred_element_type=jnp.float32)
                parts[i] = d if s == 0 else parts[i] + d
        for i in range(NCH):
            r0 = RB * i
            r1 = min(r0 + RB, N)
            inter_ref[r0:, r0:r1] += parts[i]
        th = jnp.maximum(jnp.tanh(x.astype(jnp.bfloat16) * jnp.bfloat16(0.5)),
                         zero)
        sig_ref[...] += jnp.sum(th.astype(jnp.float32), axis=1, keepdims=True)

    @pl.when(k < NK - 1)
    def _steady():
        _accum(masks_ref[...])

    @pl.when(k == NK - 1)
    def _tail():
        col = jax.lax.broadcasted_iota(jnp.int32, (N, KB), 1) + k * KB
        _accum(jnp.where(col < K, masks_ref[...], -1.0))

    @pl.when(k == NK - 1)
    def _finish():
        ri = jax.lax.broadcasted_iota(jnp.int32, (N, N), 0)
        ci = jax.lax.broadcasted_iota(jnp.int32, (N, N), 1)
        ut = inter_ref[...]
        stored = (ri // RB) >= (ci // RB)
        inter = jnp.where(stored, ut, ut.T)

        on_diag = ri == ci
        diag = jnp.where(on_diag, inter, 0.0)
        pn_col = jnp.sum(diag, axis=1, keepdims=True)
        pn_row = jnp.sum(diag, axis=0, keepdims=True)

        labels = labels_ref[...]
        m = jnp.max(labels, axis=1, keepdims=True)
        e = jnp.exp(labels - m)
        p = e / jnp.sum(e, axis=1, keepdims=True)
        li = jax.lax.broadcasted_iota(jnp.int32, (N, 19), 1)
        nms = jnp.max(jnp.where(li < 18, p, -1.0), axis=1, keepdims=True)
        nms = jnp.where(pn_col == 0.0, 0.0, nms)

        conflict = inter * (1.0 + NMS_THR) > NMS_THR * (pn_col + pn_row + 1e-6)

        s_col = nms
        s_row = jnp.sum(jnp.where(on_diag, s_col, 0.0), axis=0, keepdims=True)
        earlier = (s_row > s_col) | ((s_row == s_col) & (ci < ri))
        c_f = jnp.where(conflict & earlier, 1.0, 0.0)

        def cond(carry):
            _, changed, t = carry
            return changed & (t < N + 2)

        def body(carry):
            keep, _, t = carry
            sup = jax.lax.dot_general(
                c_f, keep, (((1,), (0,)), ((), ())),
                preferred_element_type=jnp.float32)
            new = jnp.where(sup > 0.5, 0.0, 1.0)
            changed = jnp.sum(jnp.abs(new - keep)) > 0.0
            return new, changed, t + 1

        keep0 = jnp.ones((N, 1), jnp.float32)
        keep, _, _ = jax.lax.while_loop(
            cond, body, (keep0, jnp.bool_(True), jnp.int32(0)))

        mask_scores = (0.5 * pn_col + 0.5 * sig_ref[...]) / (pn_col + 1e-6)
        out_ref[...] = nms * mask_scores * keep


def kernel(pred_labels, pred_masks):
    out = pl.pallas_call(
        _body,
        grid=(NK,),
        in_specs=[
            pl.BlockSpec((N, 19), lambda k: (0, 0)),
            pl.BlockSpec((N, KB), lambda k: (0, k)),
        ],
        out_specs=pl.BlockSpec((N, 1), lambda k: (0, 0)),
        out_shape=jax.ShapeDtypeStruct((N, 1), jnp.float32),
        scratch_shapes=[
            pltpu.VMEM((N, N), jnp.float32),
            pltpu.VMEM((N, 1), jnp.float32),
        ],
        compiler_params=pltpu.CompilerParams(
            vmem_limit_bytes=100 * 1024 * 1024),
    )(pred_labels, pred_masks)
    return out.reshape(N)
